# Initial kernel scaffold; baseline (speedup 1.0000x reference)
import jax
import jax.numpy as jnp
from jax.experimental import pallas as pl


def kernel(x, last_update, edge_index, t, msg, time_w, time_b, Wq, bq, Wk, bk, Wv, bv, We, be, Wskip, bskip):
    raise NotImplementedError("write your pallas kernel here")


# speedup vs baseline: 2.0664x; 2.0664x over previous
import functools
import math

import jax
import jax.numpy as jnp
from jax import lax
from jax.experimental import pallas as pl
from jax.experimental.pallas import tpu as pltpu

N = 10000
E = 160000
D = 256
H = 2
C = 128
HC = H * C
TDIM = 100
TPAD = 128
MSG = 256

E_PAD = 163840
N_PAD = 10240
EB = 512
NB = 1024


def _qkvs_body(x_ref, w_ref, b_ref, o_ref):
    o_ref[...] = (
        jnp.dot(x_ref[...], w_ref[...], preferred_element_type=jnp.float32)
        + b_ref[...]
    )


def _qkvs(x_pad, w_all, b_all):
    grid = (N_PAD // NB,)
    return pl.pallas_call(
        _qkvs_body,
        grid=grid,
        in_specs=[
            pl.BlockSpec((NB, D), lambda i: (i, 0)),
            pl.BlockSpec((D, 4 * HC), lambda i: (0, 0)),
            pl.BlockSpec((1, 4 * HC), lambda i: (0, 0)),
        ],
        out_specs=pl.BlockSpec((NB, 4 * HC), lambda i: (i, 0)),
        out_shape=jax.ShapeDtypeStruct((N_PAD, 4 * HC), jnp.float32),
    )(x_pad, w_all, b_all)


def _edgee_body(rt_ref, msg_ref, tw_ref, tb_ref, wet_ref, wem_ref, be_ref, o_ref):
    rt = rt_ref[0]
    ang = rt[:, None] * tw_ref[0][None, :] + tb_ref[0][None, :]
    enc = jnp.cos(ang)
    o_ref[...] = (
        jnp.dot(enc, wet_ref[...], preferred_element_type=jnp.float32)
        + jnp.dot(msg_ref[...], wem_ref[...], preferred_element_type=jnp.float32)
        + be_ref[...]
    )


def _edge_e(rel_t2, msg_pad, tw_pad, tb_pad, we_t, we_m, be):
    grid = (E_PAD // EB,)
    return pl.pallas_call(
        _edgee_body,
        grid=grid,
        in_specs=[
            pl.BlockSpec((1, EB), lambda i: (0, i)),
            pl.BlockSpec((EB, MSG), lambda i: (i, 0)),
            pl.BlockSpec((1, TPAD), lambda i: (0, 0)),
            pl.BlockSpec((1, TPAD), lambda i: (0, 0)),
            pl.BlockSpec((TPAD, HC), lambda i: (0, 0)),
            pl.BlockSpec((MSG, HC), lambda i: (0, 0)),
            pl.BlockSpec((1, HC), lambda i: (0, 0)),
        ],
        out_specs=pl.BlockSpec((EB, HC), lambda i: (i, 0)),
        out_shape=jax.ShapeDtypeStruct((E_PAD, HC), jnp.float32),
    )(rel_t2, msg_pad, tw_pad, tb_pad, we_t, we_m, be)


def _logits_body(qe_ref, ke_ref, e_ref, l0_ref, l1_ref):
    s = qe_ref[...] * (ke_ref[...] + e_ref[...])
    scale = 1.0 / math.sqrt(float(C))
    l0_ref[0] = jnp.sum(s[:, :C], axis=1) * scale
    l1_ref[0] = jnp.sum(s[:, C:], axis=1) * scale


def _logits(qe, ke, e):
    grid = (E_PAD // EB,)
    o = jax.ShapeDtypeStruct((1, E_PAD), jnp.float32)
    return pl.pallas_call(
        _logits_body,
        grid=grid,
        in_specs=[
            pl.BlockSpec((EB, HC), lambda i: (i, 0)),
            pl.BlockSpec((EB, HC), lambda i: (i, 0)),
            pl.BlockSpec((EB, HC), lambda i: (i, 0)),
        ],
        out_specs=[
            pl.BlockSpec((1, EB), lambda i: (0, i)),
            pl.BlockSpec((1, EB), lambda i: (0, i)),
        ],
        out_shape=[o, o],
    )(qe, ke, e)


def _msg_body(p0_ref, p1_ref, ve_ref, e_ref, o_ref):
    w = ve_ref[...] + e_ref[...]
    p0 = p0_ref[0]
    p1 = p1_ref[0]
    pb = jnp.concatenate(
        [jnp.broadcast_to(p0[:, None], (EB, C)),
         jnp.broadcast_to(p1[:, None], (EB, C))], axis=1)
    o_ref[...] = w * pb


def _msg_out(p0, p1, ve, e):
    grid = (E_PAD // EB,)
    return pl.pallas_call(
        _msg_body,
        grid=grid,
        in_specs=[
            pl.BlockSpec((1, EB), lambda i: (0, i)),
            pl.BlockSpec((1, EB), lambda i: (0, i)),
            pl.BlockSpec((EB, HC), lambda i: (i, 0)),
            pl.BlockSpec((EB, HC), lambda i: (i, 0)),
        ],
        out_specs=pl.BlockSpec((EB, HC), lambda i: (i, 0)),
        out_shape=jax.ShapeDtypeStruct((E_PAD, HC), jnp.float32),
    )(p0, p1, ve, e)


def _final_body(ag_ref, d0_ref, d1_ref, qkvs_ref, o_ref):
    JB = 1000
    r0 = 1.0 / (d0_ref[...] + 1e-16)
    r1 = 1.0 / (d1_ref[...] + 1e-16)
    rb = jnp.concatenate(
        [jnp.broadcast_to(r0, (JB, C)),
         jnp.broadcast_to(r1, (JB, C))], axis=1)
    o_ref[...] = ag_ref[...] * rb + qkvs_ref[...]


def _final(aggr, d0, d1, qkvs):
    JB = 1000
    grid = (N // JB,)
    return pl.pallas_call(
        _final_body,
        grid=grid,
        in_specs=[
            pl.BlockSpec((JB, HC), lambda i: (i, 0)),
            pl.BlockSpec((JB, 1), lambda i: (i, 0)),
            pl.BlockSpec((JB, 1), lambda i: (i, 0)),
            pl.BlockSpec((JB, HC), lambda i: (i, 3)),
        ],
        out_specs=pl.BlockSpec((JB, HC), lambda i: (i, 0)),
        out_shape=jax.ShapeDtypeStruct((N, HC), jnp.float32),
    )(aggr, d0, d1, qkvs)


def kernel(x, last_update, edge_index, t, msg, time_w, time_b,
           Wq, bq, Wk, bk, Wv, bv, We, be, Wskip, bskip):
    f32 = jnp.float32
    src = edge_index[0]
    dst = edge_index[1]
    src_p = jnp.pad(src, (0, E_PAD - E))
    dst_p = jnp.pad(dst, (0, E_PAD - E), constant_values=N)
    t_p = jnp.pad(t, (0, E_PAD - E))
    msg_p = jnp.pad(msg, ((0, E_PAD - E), (0, 0)))
    x_p = jnp.pad(x, ((0, N_PAD - N), (0, 0)))
    lu_p = jnp.pad(last_update, (0, N_PAD - N))

    w_all = jnp.concatenate([Wq, Wk, Wv, Wskip], axis=1)
    b_all = jnp.concatenate([bq, bk, bv, bskip])[None, :]
    tw_pad = jnp.pad(time_w, (0, TPAD - TDIM))[None, :]
    tb_pad = jnp.pad(time_b, (0, TPAD - TDIM))[None, :]
    we_t = jnp.pad(We[:TDIM], ((0, TPAD - TDIM), (0, 0)))
    we_m = We[TDIM:]
    be2 = be[None, :]

    qkvs = _qkvs(x_p, w_all, b_all)
    q = qkvs[:, 0:HC]
    k = qkvs[:, HC:2 * HC]
    v = qkvs[:, 2 * HC:3 * HC]

    rel_t = t_p - lu_p[src_p]
    rel_t2 = rel_t.reshape(1, E_PAD)

    e = _edge_e(rel_t2, msg_p, tw_pad, tb_pad, we_t, we_m, be2)

    qe = q[dst_p]
    ke = k[src_p]
    ve = v[src_p]

    l0, l1 = _logits(qe, ke, e)

    m0 = jax.ops.segment_max(l0[0], dst_p, num_segments=N_PAD)
    m1 = jax.ops.segment_max(l1[0], dst_p, num_segments=N_PAD)
    m0 = jnp.where(jnp.isfinite(m0), m0, 0.0)
    m1 = jnp.where(jnp.isfinite(m1), m1, 0.0)

    p0 = jnp.exp(l0[0] - m0[dst_p])
    p1 = jnp.exp(l1[0] - m1[dst_p])
    d0 = jax.ops.segment_sum(p0, dst_p, num_segments=N_PAD)
    d1 = jax.ops.segment_sum(p1, dst_p, num_segments=N_PAD)

    mo = _msg_out(p0.reshape(1, E_PAD), p1.reshape(1, E_PAD), ve, e)

    aggr = jax.ops.segment_sum(mo, dst_p, num_segments=N_PAD)

    out = _final(aggr[:N], d0[:N].reshape(N, 1), d1[:N].reshape(N, 1), qkvs[:N])
    return out


# speedup vs baseline: 3.2571x; 1.5762x over previous
import functools
import math

import jax
import jax.numpy as jnp
from jax import lax
from jax.experimental import pallas as pl
from jax.experimental.pallas import tpu as pltpu
from jax.experimental.pallas import tpu_sc as plsc

NC = 2
NS = 16
NW = NC * NS
L = 16

N = 10000
E = 160000
D = 256
H = 2
C = 128
HC = H * C
TDIM = 100
TPAD = 128
MSG = 256

E_PAD = 163840
N_PAD = 10240
EB = 512
NB = 1024


def _qkvs_body(x_ref, w_ref, b_ref, o_ref):
    o_ref[...] = (
        jnp.dot(x_ref[...], w_ref[...], preferred_element_type=jnp.float32)
        + b_ref[...]
    )


def _qkvs(x_pad, w_all, b_all):
    grid = (N_PAD // NB,)
    return pl.pallas_call(
        _qkvs_body,
        grid=grid,
        in_specs=[
            pl.BlockSpec((NB, D), lambda i: (i, 0)),
            pl.BlockSpec((D, 4 * HC), lambda i: (0, 0)),
            pl.BlockSpec((1, 4 * HC), lambda i: (0, 0)),
        ],
        out_specs=pl.BlockSpec((NB, 4 * HC), lambda i: (i, 0)),
        out_shape=jax.ShapeDtypeStruct((N_PAD, 4 * HC), jnp.float32),
    )(x_pad, w_all, b_all)


def _edgee_body(rt_ref, msg_ref, tw_ref, tb_ref, wet_ref, wem_ref, be_ref, o_ref):
    rt = rt_ref[0]
    ang = rt[:, None] * tw_ref[0][None, :] + tb_ref[0][None, :]
    enc = jnp.cos(ang)
    o_ref[...] = (
        jnp.dot(enc, wet_ref[...], preferred_element_type=jnp.float32)
        + jnp.dot(msg_ref[...], wem_ref[...], preferred_element_type=jnp.float32)
        + be_ref[...]
    )


def _edge_e(rel_t2, msg_pad, tw_pad, tb_pad, we_t, we_m, be):
    grid = (E_PAD // EB,)
    return pl.pallas_call(
        _edgee_body,
        grid=grid,
        in_specs=[
            pl.BlockSpec((1, EB), lambda i: (0, i)),
            pl.BlockSpec((EB, MSG), lambda i: (i, 0)),
            pl.BlockSpec((1, TPAD), lambda i: (0, 0)),
            pl.BlockSpec((1, TPAD), lambda i: (0, 0)),
            pl.BlockSpec((TPAD, HC), lambda i: (0, 0)),
            pl.BlockSpec((MSG, HC), lambda i: (0, 0)),
            pl.BlockSpec((1, HC), lambda i: (0, 0)),
        ],
        out_specs=pl.BlockSpec((EB, HC), lambda i: (i, 0)),
        out_shape=jax.ShapeDtypeStruct((E_PAD, HC), jnp.float32),
    )(rel_t2, msg_pad, tw_pad, tb_pad, we_t, we_m, be)


def _logits_body(qe_ref, ke_ref, e_ref, l0_ref, l1_ref):
    s = qe_ref[...] * (ke_ref[...] + e_ref[...])
    scale = 1.0 / math.sqrt(float(C))
    l0_ref[0] = jnp.sum(s[:, :C], axis=1) * scale
    l1_ref[0] = jnp.sum(s[:, C:], axis=1) * scale


def _logits(qe, ke, e):
    grid = (E_PAD // EB,)
    o = jax.ShapeDtypeStruct((1, E_PAD), jnp.float32)
    return pl.pallas_call(
        _logits_body,
        grid=grid,
        in_specs=[
            pl.BlockSpec((EB, HC), lambda i: (i, 0)),
            pl.BlockSpec((EB, HC), lambda i: (i, 0)),
            pl.BlockSpec((EB, HC), lambda i: (i, 0)),
        ],
        out_specs=[
            pl.BlockSpec((1, EB), lambda i: (0, i)),
            pl.BlockSpec((1, EB), lambda i: (0, i)),
        ],
        out_shape=[o, o],
    )(qe, ke, e)


def _msg_body(p0_ref, p1_ref, ve_ref, e_ref, o_ref):
    w = ve_ref[...] + e_ref[...]
    p0 = p0_ref[0]
    p1 = p1_ref[0]
    pb = jnp.concatenate(
        [jnp.broadcast_to(p0[:, None], (EB, C)),
         jnp.broadcast_to(p1[:, None], (EB, C))], axis=1)
    o_ref[...] = w * pb


def _msg_out(p0, p1, ve, e):
    grid = (E_PAD // EB,)
    return pl.pallas_call(
        _msg_body,
        grid=grid,
        in_specs=[
            pl.BlockSpec((1, EB), lambda i: (0, i)),
            pl.BlockSpec((1, EB), lambda i: (0, i)),
            pl.BlockSpec((EB, HC), lambda i: (i, 0)),
            pl.BlockSpec((EB, HC), lambda i: (i, 0)),
        ],
        out_specs=pl.BlockSpec((EB, HC), lambda i: (i, 0)),
        out_shape=jax.ShapeDtypeStruct((E_PAD, HC), jnp.float32),
    )(p0, p1, ve, e)


def _final_body(ag_ref, d0_ref, d1_ref, qkvs_ref, o_ref):
    JB = 1000
    r0 = 1.0 / (d0_ref[...] + 1e-16)
    r1 = 1.0 / (d1_ref[...] + 1e-16)
    rb = jnp.concatenate(
        [jnp.broadcast_to(r0, (JB, C)),
         jnp.broadcast_to(r1, (JB, C))], axis=1)
    o_ref[...] = ag_ref[...] * rb + qkvs_ref[...]


def _final(aggr, d0, d1, qkvs):
    JB = 1000
    grid = (N // JB,)
    return pl.pallas_call(
        _final_body,
        grid=grid,
        in_specs=[
            pl.BlockSpec((JB, HC), lambda i: (i, 0)),
            pl.BlockSpec((JB, 1), lambda i: (i, 0)),
            pl.BlockSpec((JB, 1), lambda i: (i, 0)),
            pl.BlockSpec((JB, HC), lambda i: (i, 3)),
        ],
        out_specs=pl.BlockSpec((JB, HC), lambda i: (i, 0)),
        out_shape=jax.ShapeDtypeStruct((N, HC), jnp.float32),
    )(aggr, d0, d1, qkvs)


CH = 128
ROWS2 = E_PAD // CH
CPT = ROWS2 // NW
_MESH = dict(core_axis_name="c", subcore_axis_name="s")


def _sc_relt(src2, t2, lu_p):
    @functools.partial(
        pl.kernel,
        out_type=jax.ShapeDtypeStruct((ROWS2, CH), jnp.float32),
        mesh=plsc.VectorSubcoreMesh(**_MESH),
        scratch_types=[
            pltpu.VMEM((CPT, CH), jnp.int32),
            pltpu.VMEM((CPT, CH), jnp.float32),
            pltpu.VMEM((CH,), jnp.float32),
            pltpu.VMEM((CPT, CH), jnp.float32),
            pltpu.SemaphoreType.DMA,
        ],
    )
    def k(src_hbm, t_hbm, lu_hbm, out_hbm, idx_v, t_v, row_v, o_v, sem):
        wid = lax.axis_index("s") * NC + lax.axis_index("c")
        base = wid * CPT
        pltpu.sync_copy(src_hbm.at[pl.ds(base, CPT)], idx_v)
        pltpu.sync_copy(t_hbm.at[pl.ds(base, CPT)], t_v)

        def chunk(j, carry):
            pltpu.async_copy(lu_hbm.at[idx_v.at[j]], row_v, sem).wait()

            def vec(i, c2):
                o_v[j, pl.ds(i * L, L)] = (
                    t_v[j, pl.ds(i * L, L)] - row_v[pl.ds(i * L, L)])
                return c2

            return lax.fori_loop(0, CH // L, vec, carry)

        lax.fori_loop(0, CPT, chunk, 0)
        pltpu.sync_copy(o_v, out_hbm.at[pl.ds(base, CPT)])

    return k(src2, t2, lu_p)


def _sc_gather3(q, kk, v, src2, dst2):
    ot = jax.ShapeDtypeStruct((E_PAD, HC), jnp.float32)

    @functools.partial(
        pl.kernel,
        out_type=[ot, ot, ot],
        mesh=plsc.VectorSubcoreMesh(**_MESH),
        scratch_types=[
            pltpu.VMEM((CPT, CH), jnp.int32),
            pltpu.VMEM((CPT, CH), jnp.int32),
            pltpu.VMEM((CH, HC), jnp.float32),
            pltpu.VMEM((CH, HC), jnp.float32),
            pltpu.SemaphoreType.DMA,
            pltpu.SemaphoreType.DMA,
        ],
    )
    def k(q_hbm, k_hbm, v_hbm, src_hbm, dst_hbm,
          qe_hbm, ke_hbm, ve_hbm, sidx, didx, bufa, bufb, sema, semb):
        wid = lax.axis_index("s") * NC + lax.axis_index("c")
        base = wid * CPT
        pltpu.sync_copy(src_hbm.at[pl.ds(base, CPT)], sidx)
        pltpu.sync_copy(dst_hbm.at[pl.ds(base, CPT)], didx)

        for table, idx, out in ((q_hbm, didx, qe_hbm),
                                (k_hbm, sidx, ke_hbm),
                                (v_hbm, sidx, ve_hbm)):
            pltpu.async_copy(table.at[idx.at[0]], bufa, sema)

            def pair(jj, carry, table=table, idx=idx, out=out):
                j0 = 2 * jj
                pltpu.async_copy(table.at[idx.at[j0 + 1]], bufb, semb)
                pltpu.make_async_copy(table.at[idx.at[j0]], bufa, sema).wait()
                pltpu.sync_copy(bufa, out.at[pl.ds((base + j0) * CH, CH)])

                @pl.when(jj < CPT // 2 - 1)
                def _():
                    pltpu.async_copy(table.at[idx.at[j0 + 2]], bufa, sema)

                pltpu.make_async_copy(table.at[idx.at[j0]], bufb, semb).wait()
                pltpu.sync_copy(bufb, out.at[pl.ds((base + j0 + 1) * CH, CH)])
                return carry

            lax.fori_loop(0, CPT // 2, pair, 0)

    return k(q, kk, v, src2, dst2)


def kernel(x, last_update, edge_index, t, msg, time_w, time_b,
           Wq, bq, Wk, bk, Wv, bv, We, be, Wskip, bskip):
    f32 = jnp.float32
    src = edge_index[0]
    dst = edge_index[1]
    src_p = jnp.pad(src, (0, E_PAD - E))
    dst_p = jnp.pad(dst, (0, E_PAD - E), constant_values=N)
    t_p = jnp.pad(t, (0, E_PAD - E))
    msg_p = jnp.pad(msg, ((0, E_PAD - E), (0, 0)))
    x_p = jnp.pad(x, ((0, N_PAD - N), (0, 0)))
    lu_p = jnp.pad(last_update, (0, N_PAD - N))

    w_all = jnp.concatenate([Wq, Wk, Wv, Wskip], axis=1)
    b_all = jnp.concatenate([bq, bk, bv, bskip])[None, :]
    tw_pad = jnp.pad(time_w, (0, TPAD - TDIM))[None, :]
    tb_pad = jnp.pad(time_b, (0, TPAD - TDIM))[None, :]
    we_t = jnp.pad(We[:TDIM], ((0, TPAD - TDIM), (0, 0)))
    we_m = We[TDIM:]
    be2 = be[None, :]

    qkvs = _qkvs(x_p, w_all, b_all)
    q = qkvs[:, 0:HC]
    k = qkvs[:, HC:2 * HC]
    v = qkvs[:, 2 * HC:3 * HC]

    src2 = src_p.reshape(ROWS2, CH)
    dst2 = dst_p.reshape(ROWS2, CH)
    t2 = t_p.reshape(ROWS2, CH)
    rel_t2 = _sc_relt(src2, t2, lu_p).reshape(1, E_PAD)

    e = _edge_e(rel_t2, msg_p, tw_pad, tb_pad, we_t, we_m, be2)

    qe, ke, ve = _sc_gather3(q, k, v, src2, dst2)

    l0, l1 = _logits(qe, ke, e)

    m0 = jax.ops.segment_max(l0[0], dst_p, num_segments=N_PAD)
    m1 = jax.ops.segment_max(l1[0], dst_p, num_segments=N_PAD)
    m0 = jnp.where(jnp.isfinite(m0), m0, 0.0)
    m1 = jnp.where(jnp.isfinite(m1), m1, 0.0)

    p0 = jnp.exp(l0[0] - m0[dst_p])
    p1 = jnp.exp(l1[0] - m1[dst_p])
    d0 = jax.ops.segment_sum(p0, dst_p, num_segments=N_PAD)
    d1 = jax.ops.segment_sum(p1, dst_p, num_segments=N_PAD)

    mo = _msg_out(p0.reshape(1, E_PAD), p1.reshape(1, E_PAD), ve, e)

    aggr = jax.ops.segment_sum(mo, dst_p, num_segments=N_PAD)

    out = _final(aggr[:N], d0[:N].reshape(N, 1), d1[:N].reshape(N, 1), qkvs[:N])
    return out


# speedup vs baseline: 10.4520x; 3.2090x over previous
import functools
import math

import jax
import jax.numpy as jnp
from jax import lax
from jax.experimental import pallas as pl
from jax.experimental.pallas import tpu as pltpu
from jax.experimental.pallas import tpu_sc as plsc

NC = 2
NS = 16
NW = NC * NS
L = 16

N = 10000
E = 160000
D = 256
H = 2
C = 128
HC = H * C
TDIM = 100
TPAD = 128
MSG = 256

E_PAD = 163840
N_PAD = 10240
EB = 512
NB = 1024


def _qkvs_body(x_ref, w_ref, b_ref, o_ref):
    o_ref[...] = (
        jnp.dot(x_ref[...], w_ref[...], preferred_element_type=jnp.float32)
        + b_ref[...]
    )


def _qkvs(x_pad, w_all, b_all):
    grid = (N_PAD // NB,)
    return pl.pallas_call(
        _qkvs_body,
        grid=grid,
        in_specs=[
            pl.BlockSpec((NB, D), lambda i: (i, 0)),
            pl.BlockSpec((D, 4 * HC), lambda i: (0, 0)),
            pl.BlockSpec((1, 4 * HC), lambda i: (0, 0)),
        ],
        out_specs=pl.BlockSpec((NB, 4 * HC), lambda i: (i, 0)),
        out_shape=jax.ShapeDtypeStruct((N_PAD, 4 * HC), jnp.float32),
    )(x_pad, w_all, b_all)


def _edgee_body(rt_ref, msg_ref, tw_ref, tb_ref, wet_ref, wem_ref, be_ref, o_ref):
    rt = rt_ref[0]
    ang = rt[:, None] * tw_ref[0][None, :] + tb_ref[0][None, :]
    enc = jnp.cos(ang)
    o_ref[...] = (
        jnp.dot(enc, wet_ref[...], preferred_element_type=jnp.float32)
        + jnp.dot(msg_ref[...], wem_ref[...], preferred_element_type=jnp.float32)
        + be_ref[...]
    )


def _edge_e(rel_t2, msg_pad, tw_pad, tb_pad, we_t, we_m, be):
    grid = (E_PAD // EB,)
    return pl.pallas_call(
        _edgee_body,
        grid=grid,
        in_specs=[
            pl.BlockSpec((1, EB), lambda i: (0, i)),
            pl.BlockSpec((EB, MSG), lambda i: (i, 0)),
            pl.BlockSpec((1, TPAD), lambda i: (0, 0)),
            pl.BlockSpec((1, TPAD), lambda i: (0, 0)),
            pl.BlockSpec((TPAD, HC), lambda i: (0, 0)),
            pl.BlockSpec((MSG, HC), lambda i: (0, 0)),
            pl.BlockSpec((1, HC), lambda i: (0, 0)),
        ],
        out_specs=pl.BlockSpec((EB, HC), lambda i: (i, 0)),
        out_shape=jax.ShapeDtypeStruct((E_PAD, HC), jnp.float32),
    )(rel_t2, msg_pad, tw_pad, tb_pad, we_t, we_m, be)


def _logits_body(qe_ref, ke_ref, e_ref, l_ref):
    s = qe_ref[...] * (ke_ref[...] + e_ref[...])
    scale = 1.0 / math.sqrt(float(C))
    l0 = jnp.sum(s[:, :C], axis=1) * scale
    l1 = jnp.sum(s[:, C:], axis=1) * scale
    l_ref[...] = jnp.stack([l0, l1])


def _logits(qe, ke, e):
    grid = (E_PAD // EB,)
    return pl.pallas_call(
        _logits_body,
        grid=grid,
        in_specs=[
            pl.BlockSpec((EB, HC), lambda i: (i, 0)),
            pl.BlockSpec((EB, HC), lambda i: (i, 0)),
            pl.BlockSpec((EB, HC), lambda i: (i, 0)),
        ],
        out_specs=pl.BlockSpec((2, EB), lambda i: (0, i)),
        out_shape=jax.ShapeDtypeStruct((2, E_PAD), jnp.float32),
    )(qe, ke, e)


def _msg_body(p0_ref, p1_ref, ve_ref, e_ref, o_ref):
    w = ve_ref[...] + e_ref[...]
    p0 = p0_ref[0]
    p1 = p1_ref[0]
    pb = jnp.concatenate(
        [jnp.broadcast_to(p0[:, None], (EB, C)),
         jnp.broadcast_to(p1[:, None], (EB, C))], axis=1)
    o_ref[...] = w * pb


def _msg_out(p0, p1, ve, e):
    grid = (E_PAD // EB,)
    return pl.pallas_call(
        _msg_body,
        grid=grid,
        in_specs=[
            pl.BlockSpec((1, EB), lambda i: (0, i)),
            pl.BlockSpec((1, EB), lambda i: (0, i)),
            pl.BlockSpec((EB, HC), lambda i: (i, 0)),
            pl.BlockSpec((EB, HC), lambda i: (i, 0)),
        ],
        out_specs=pl.BlockSpec((EB, HC), lambda i: (i, 0)),
        out_shape=jax.ShapeDtypeStruct((E_PAD, HC), jnp.float32),
    )(p0, p1, ve, e)


def _final_body(ag_ref, d0_ref, d1_ref, qkvs_ref, o_ref):
    JB = 1000
    r0 = 1.0 / (d0_ref[...] + 1e-16)
    r1 = 1.0 / (d1_ref[...] + 1e-16)
    rb = jnp.concatenate(
        [jnp.broadcast_to(r0, (JB, C)),
         jnp.broadcast_to(r1, (JB, C))], axis=1)
    o_ref[...] = ag_ref[...] * rb + qkvs_ref[...]


def _final(aggr, d0, d1, qkvs):
    JB = 1000
    grid = (N // JB,)
    return pl.pallas_call(
        _final_body,
        grid=grid,
        in_specs=[
            pl.BlockSpec((JB, HC), lambda i: (i, 0)),
            pl.BlockSpec((JB, 1), lambda i: (i, 0)),
            pl.BlockSpec((JB, 1), lambda i: (i, 0)),
            pl.BlockSpec((JB, HC), lambda i: (i, 3)),
        ],
        out_specs=pl.BlockSpec((JB, HC), lambda i: (i, 0)),
        out_shape=jax.ShapeDtypeStruct((N, HC), jnp.float32),
    )(aggr, d0, d1, qkvs)


CH = 128
ROWS2 = E_PAD // CH
CPT = ROWS2 // NW
_MESH = dict(core_axis_name="c", subcore_axis_name="s")


def _sc_relt(src2, t2, lu_p):
    @functools.partial(
        pl.kernel,
        out_type=jax.ShapeDtypeStruct((ROWS2, CH), jnp.float32),
        mesh=plsc.VectorSubcoreMesh(**_MESH),
        scratch_types=[
            pltpu.VMEM((CPT, CH), jnp.int32),
            pltpu.VMEM((CPT, CH), jnp.float32),
            pltpu.VMEM((CH,), jnp.float32),
            pltpu.VMEM((CPT, CH), jnp.float32),
            pltpu.SemaphoreType.DMA,
        ],
    )
    def k(src_hbm, t_hbm, lu_hbm, out_hbm, idx_v, t_v, row_v, o_v, sem):
        wid = lax.axis_index("s") * NC + lax.axis_index("c")
        base = wid * CPT
        pltpu.sync_copy(src_hbm.at[pl.ds(base, CPT)], idx_v)
        pltpu.sync_copy(t_hbm.at[pl.ds(base, CPT)], t_v)

        def chunk(j, carry):
            pltpu.async_copy(lu_hbm.at[idx_v.at[j]], row_v, sem).wait()

            def vec(i, c2):
                o_v[j, pl.ds(i * L, L)] = (
                    t_v[j, pl.ds(i * L, L)] - row_v[pl.ds(i * L, L)])
                return c2

            return lax.fori_loop(0, CH // L, vec, carry)

        lax.fori_loop(0, CPT, chunk, 0)
        pltpu.sync_copy(o_v, out_hbm.at[pl.ds(base, CPT)])

    return k(src2, t2, lu_p)


def _sc_gather3(q, kk, v, src2, dst2):
    ot = jax.ShapeDtypeStruct((E_PAD, HC), jnp.float32)

    @functools.partial(
        pl.kernel,
        out_type=[ot, ot, ot],
        mesh=plsc.VectorSubcoreMesh(**_MESH),
        scratch_types=[
            pltpu.VMEM((CPT, CH), jnp.int32),
            pltpu.VMEM((CPT, CH), jnp.int32),
            pltpu.VMEM((CH, HC), jnp.float32),
            pltpu.VMEM((CH, HC), jnp.float32),
            pltpu.SemaphoreType.DMA,
            pltpu.SemaphoreType.DMA,
        ],
    )
    def k(q_hbm, k_hbm, v_hbm, src_hbm, dst_hbm,
          qe_hbm, ke_hbm, ve_hbm, sidx, didx, bufa, bufb, sema, semb):
        wid = lax.axis_index("s") * NC + lax.axis_index("c")
        base = wid * CPT
        pltpu.sync_copy(src_hbm.at[pl.ds(base, CPT)], sidx)
        pltpu.sync_copy(dst_hbm.at[pl.ds(base, CPT)], didx)

        for table, idx, out in ((q_hbm, didx, qe_hbm),
                                (k_hbm, sidx, ke_hbm),
                                (v_hbm, sidx, ve_hbm)):
            pltpu.async_copy(table.at[idx.at[0]], bufa, sema)

            def pair(jj, carry, table=table, idx=idx, out=out):
                j0 = 2 * jj
                pltpu.async_copy(table.at[idx.at[j0 + 1]], bufb, semb)
                pltpu.make_async_copy(table.at[idx.at[j0]], bufa, sema).wait()
                pltpu.sync_copy(bufa, out.at[pl.ds((base + j0) * CH, CH)])

                @pl.when(jj < CPT // 2 - 1)
                def _():
                    pltpu.async_copy(table.at[idx.at[j0 + 2]], bufa, sema)

                pltpu.make_async_copy(table.at[idx.at[j0]], bufb, semb).wait()
                pltpu.sync_copy(bufb, out.at[pl.ds((base + j0 + 1) * CH, CH)])
                return carry

            lax.fori_loop(0, CPT // 2, pair, 0)

    return k(q, kk, v, src2, dst2)


EPT = E_PAD // NS
NPS = N_PAD // NS
FMIN = -3.0e38


def _sc_softmax(l2, dst2):
    @functools.partial(
        pl.kernel,
        out_type=[jax.ShapeDtypeStruct((2, E_PAD), jnp.float32),
                  jax.ShapeDtypeStruct((2, N_PAD), jnp.float32)],
        mesh=plsc.VectorSubcoreMesh(**_MESH),
        compiler_params=pltpu.CompilerParams(needs_layout_passes=False),
        scratch_types=[
            pltpu.VMEM((EPT,), jnp.float32),
            pltpu.VMEM((EPT,), jnp.int32),
            pltpu.VMEM((N_PAD,), jnp.float32),
            pltpu.VMEM((N_PAD,), jnp.float32),
            pltpu.VMEM((EPT,), jnp.float32),
            pltpu.VMEM((NS, NPS), jnp.float32),
            pltpu.VMEM_SHARED((NS, N_PAD), jnp.float32),
            pltpu.VMEM_SHARED((N_PAD,), jnp.float32),
        ],
    )
    def k(l_hbm, dst_hbm, p_hbm, d_hbm,
          l_v, dv_v, tab, dtab, p_v, cmb, shtabs, shm):
        c = lax.axis_index("c")
        sid = lax.axis_index("s")
        base = sid * EPT
        pltpu.sync_copy(l_hbm.at[c, pl.ds(base, EPT)], l_v)
        pltpu.sync_copy(dst_hbm.at[pl.ds(base, EPT)], dv_v)

        def init(i, carry):
            tab[pl.ds(i * L, L)] = jnp.full((L,), FMIN, jnp.float32)
            dtab[pl.ds(i * L, L)] = jnp.zeros((L,), jnp.float32)
            return carry
        lax.fori_loop(0, N_PAD // L, init, 0)

        def smax(i, carry):
            dv = dv_v[pl.ds(i * L, L)]
            lv = l_v[pl.ds(i * L, L)]
            need = lv > plsc.load_gather(tab, [dv])

            def wcond(need):
                cnt = plsc.all_reduce_population_count(need)
                return lax.reduce_max(cnt, (0,)) > 0

            def wbody(need):
                plsc.store_scatter(tab, [dv], lv, mask=need)
                return lv > plsc.load_gather(tab, [dv])

            lax.while_loop(wcond, wbody, need)
            return carry
        lax.fori_loop(0, EPT // L, smax, 0)

        pltpu.sync_copy(tab, shtabs.at[sid])
        plsc.subcore_barrier()
        pltpu.sync_copy(shtabs.at[pl.ds(0, NS), pl.ds(sid * NPS, NPS)], cmb)

        def cmax(i, carry):
            acc = cmb[0, pl.ds(i * L, L)]
            for r in range(1, NS):
                acc = jnp.maximum(acc, cmb[r, pl.ds(i * L, L)])
            tab[pl.ds(sid * NPS + i * L, L)] = acc
            return carry
        lax.fori_loop(0, NPS // L, cmax, 0)
        pltpu.sync_copy(tab.at[pl.ds(sid * NPS, NPS)],
                        shm.at[pl.ds(sid * NPS, NPS)])
        plsc.subcore_barrier()
        pltpu.sync_copy(shm, tab)

        def pexp(i, carry):
            dv = dv_v[pl.ds(i * L, L)]
            lv = l_v[pl.ds(i * L, L)]
            mv = plsc.load_gather(tab, [dv])
            pv = jnp.exp(lv - mv)
            p_v[pl.ds(i * L, L)] = pv
            plsc.addupdate_scatter(dtab, [dv], pv)
            return carry
        lax.fori_loop(0, EPT // L, pexp, 0)
        pltpu.sync_copy(p_v, p_hbm.at[c, pl.ds(base, EPT)])

        plsc.subcore_barrier()
        pltpu.sync_copy(dtab, shtabs.at[sid])
        plsc.subcore_barrier()
        pltpu.sync_copy(shtabs.at[pl.ds(0, NS), pl.ds(sid * NPS, NPS)], cmb)

        def cadd(i, carry):
            acc = cmb[0, pl.ds(i * L, L)]
            for r in range(1, NS):
                acc = acc + cmb[r, pl.ds(i * L, L)]
            dtab[pl.ds(i * L, L)] = acc
            return carry
        lax.fori_loop(0, NPS // L, cadd, 0)
        pltpu.sync_copy(dtab.at[pl.ds(0, NPS)],
                        d_hbm.at[c, pl.ds(sid * NPS, NPS)])

    return k(l2, dst2)


def _sc_aggr(mo, dst2):
    CPT2 = EPT // CH

    @functools.partial(
        pl.kernel,
        out_type=jax.ShapeDtypeStruct((N_PAD, HC), jnp.float32),
        mesh=plsc.VectorSubcoreMesh(**_MESH),
        scratch_types=[
            pltpu.VMEM((CPT2, CH), jnp.int32),
            pltpu.VMEM((CH, C), jnp.float32),
            pltpu.VMEM((CH, C), jnp.float32),
            pltpu.VMEM_SHARED((N_PAD, C), jnp.float32),
            pltpu.SemaphoreType.DMA,
            pltpu.SemaphoreType.DMA,
        ],
    )
    def k(mo_hbm, dst_hbm, out_hbm, didx, bufa, bufb, shacc, sema, semb):
        c = lax.axis_index("c")
        sid = lax.axis_index("s")
        row0 = sid * CPT2
        pltpu.sync_copy(dst_hbm.at[pl.ds(row0, CPT2)], didx)

        def z(i, carry):
            r = i // (C // L)
            col = (i % (C // L)) * L
            bufa[r, pl.ds(col, L)] = jnp.zeros((L,), jnp.float32)
            return carry
        lax.fori_loop(0, CH * C // L, z, 0)
        for b in range(NPS // CH):
            pltpu.sync_copy(bufa, shacc.at[pl.ds(sid * NPS + b * CH, CH)])
        plsc.subcore_barrier()

        ecol = c * C
        pltpu.async_copy(
            mo_hbm.at[pl.ds(row0 * CH, CH), pl.ds(ecol, C)], bufa, sema)

        def pair(jj, carry):
            j0 = 2 * jj
            pltpu.async_copy(
                mo_hbm.at[pl.ds((row0 + j0 + 1) * CH, CH), pl.ds(ecol, C)],
                bufb, semb)
            pltpu.make_async_copy(
                mo_hbm.at[pl.ds(0, CH), pl.ds(ecol, C)], bufa, sema).wait()
            pltpu.sync_copy(bufa, shacc.at[didx.at[j0]], add=True)

            @pl.when(jj < CPT2 // 2 - 1)
            def _():
                pltpu.async_copy(
                    mo_hbm.at[pl.ds((row0 + j0 + 2) * CH, CH), pl.ds(ecol, C)],
                    bufa, sema)

            pltpu.make_async_copy(
                mo_hbm.at[pl.ds(0, CH), pl.ds(ecol, C)], bufb, semb).wait()
            pltpu.sync_copy(bufb, shacc.at[didx.at[j0 + 1]], add=True)
            return carry
        lax.fori_loop(0, CPT2 // 2, pair, 0)

        plsc.subcore_barrier()
        pltpu.sync_copy(shacc.at[pl.ds(sid * NPS, NPS)],
                        out_hbm.at[pl.ds(sid * NPS, NPS), pl.ds(ecol, C)])

    return k(mo, dst2)


def kernel(x, last_update, edge_index, t, msg, time_w, time_b,
           Wq, bq, Wk, bk, Wv, bv, We, be, Wskip, bskip):
    f32 = jnp.float32
    src = edge_index[0]
    dst = edge_index[1]
    src_p = jnp.pad(src, (0, E_PAD - E))
    dst_p = jnp.pad(dst, (0, E_PAD - E), constant_values=N)
    t_p = jnp.pad(t, (0, E_PAD - E))
    msg_p = jnp.pad(msg, ((0, E_PAD - E), (0, 0)))
    x_p = jnp.pad(x, ((0, N_PAD - N), (0, 0)))
    lu_p = jnp.pad(last_update, (0, N_PAD - N))

    w_all = jnp.concatenate([Wq, Wk, Wv, Wskip], axis=1)
    b_all = jnp.concatenate([bq, bk, bv, bskip])[None, :]
    tw_pad = jnp.pad(time_w, (0, TPAD - TDIM))[None, :]
    tb_pad = jnp.pad(time_b, (0, TPAD - TDIM))[None, :]
    we_t = jnp.pad(We[:TDIM], ((0, TPAD - TDIM), (0, 0)))
    we_m = We[TDIM:]
    be2 = be[None, :]

    qkvs = _qkvs(x_p, w_all, b_all)
    q = qkvs[:, 0:HC]
    k = qkvs[:, HC:2 * HC]
    v = qkvs[:, 2 * HC:3 * HC]

    src2 = src_p.reshape(ROWS2, CH)
    dst2 = dst_p.reshape(ROWS2, CH)
    t2 = t_p.reshape(ROWS2, CH)
    rel_t2 = _sc_relt(src2, t2, lu_p).reshape(1, E_PAD)

    e = _edge_e(rel_t2, msg_p, tw_pad, tb_pad, we_t, we_m, be2)

    qe, ke, ve = _sc_gather3(q, k, v, src2, dst2)

    l = _logits(qe, ke, e)

    p, d = _sc_softmax(l, dst_p)

    mo = _msg_out(p[0:1], p[1:2], ve, e)

    aggr = _sc_aggr(mo, dst2)

    out = _final(aggr[:N], d[0, :N].reshape(N, 1), d[1, :N].reshape(N, 1),
                 qkvs[:N])
    return out
